# Initial kernel scaffold; baseline (speedup 1.0000x reference)
import jax
import jax.numpy as jnp
from jax.experimental import pallas as pl


def kernel(seq_len, embedding_weight):
    raise NotImplementedError("write your pallas kernel here")


# speedup vs baseline: 1.5119x; 1.5119x over previous
import functools

import jax
import jax.numpy as jnp
from jax import lax
from jax.experimental import pallas as pl
from jax.experimental.pallas import tpu as pltpu
from jax.experimental.pallas import tpu_sc as plsc

MAX_ROWS = 8192
D = 1024

NC = 2
NS = 16
NW = NC * NS
B_PER_W = MAX_ROWS // NW
CHUNK = 64
N_CHUNKS = B_PER_W // CHUNK

_mesh = plsc.VectorSubcoreMesh(core_axis_name="c", subcore_axis_name="s")


@functools.partial(
    pl.kernel,
    mesh=_mesh,
    out_type=jax.ShapeDtypeStruct((MAX_ROWS, D), jnp.float32),
    scratch_types=[
        pltpu.VMEM((B_PER_W,), jnp.int32),
        pltpu.VMEM((CHUNK, D), jnp.float32),
        pltpu.SemaphoreType.DMA,
    ],
)
def _gather_rows(table_hbm, idx_hbm, out_hbm, idx_v, rows_v, sem):
    wid = lax.axis_index("s") * NC + lax.axis_index("c")
    base = wid * B_PER_W
    pltpu.sync_copy(idx_hbm.at[pl.ds(base, B_PER_W)], idx_v)
    for c in range(N_CHUNKS):
        pltpu.async_copy(
            table_hbm.at[idx_v.at[pl.ds(c * CHUNK, CHUNK)]], rows_v, sem
        ).wait()
        pltpu.sync_copy(rows_v, out_hbm.at[pl.ds(base + c * CHUNK, CHUNK)])


def kernel(seq_len, embedding_weight):
    n = embedding_weight.shape[0]
    last = jnp.asarray(seq_len, dtype=jnp.int32) - 1
    idx = jnp.minimum(jnp.arange(n, dtype=jnp.int32), last)
    out = _gather_rows(embedding_weight, idx)
    return out[None, :, :]


# speedup vs baseline: 1.5404x; 1.0189x over previous
import functools

import jax
import jax.numpy as jnp
from jax import lax
from jax.experimental import pallas as pl
from jax.experimental.pallas import tpu as pltpu
from jax.experimental.pallas import tpu_sc as plsc

MAX_ROWS = 8192
D = 1024

NC = 2
NS = 16
NW = NC * NS
B_PER_W = MAX_ROWS // NW
CHUNK = 32
N_CHUNKS = B_PER_W // CHUNK

_mesh = plsc.VectorSubcoreMesh(core_axis_name="c", subcore_axis_name="s")


@functools.partial(
    pl.kernel,
    mesh=_mesh,
    out_type=jax.ShapeDtypeStruct((MAX_ROWS, D), jnp.float32),
    scratch_types=[
        pltpu.VMEM((B_PER_W,), jnp.int32),
        pltpu.VMEM((CHUNK, D), jnp.float32),
        pltpu.VMEM((CHUNK, D), jnp.float32),
        pltpu.SemaphoreType.DMA,
        pltpu.SemaphoreType.DMA,
    ],
)
def _gather_rows(table_hbm, idx_hbm, out_hbm, idx_v, buf0, buf1, sem0, sem1):
    wid = lax.axis_index("s") * NC + lax.axis_index("c")
    base = wid * B_PER_W
    pltpu.sync_copy(idx_hbm.at[pl.ds(base, B_PER_W)], idx_v)
    bufs = (buf0, buf1)
    sems = (sem0, sem1)

    def _start_gather(c):
        return pltpu.async_copy(
            table_hbm.at[idx_v.at[pl.ds(c * CHUNK, CHUNK)]],
            bufs[c % 2], sems[c % 2],
        )

    copies = [None] * N_CHUNKS
    copies[0] = _start_gather(0)
    for c in range(N_CHUNKS):
        if c + 1 < N_CHUNKS:
            copies[c + 1] = _start_gather(c + 1)
        copies[c].wait()
        pltpu.sync_copy(bufs[c % 2], out_hbm.at[pl.ds(base + c * CHUNK, CHUNK)])


def kernel(seq_len, embedding_weight):
    n = embedding_weight.shape[0]
    last = jnp.asarray(seq_len, dtype=jnp.int32) - 1
    idx = jnp.minimum(jnp.arange(n, dtype=jnp.int32), last)
    out = _gather_rows(embedding_weight, idx)
    return out[None, :, :]
